# Initial kernel scaffold; baseline (speedup 1.0000x reference)
import functools

import jax
import jax.numpy as jnp
from jax import lax
from jax.experimental import pallas as pl
from jax.experimental.pallas import tpu as pltpu
from jax.experimental.pallas import tpu_sc as plsc

N_NODES = 10000
E_TOT = 320000
HEADS = 8
D_HEAD = 16
D_OUT = 128
T_DIM = 32
D_EDGE = 16

NC = 2
NS = 16
NW = NC * NS
EPW = E_TOT // NW
EB = 80
NBATCH = EPW // EB
RPS = N_NODES // NS

NODE_BLK = 400
EDGE_BLK = 2000


def _perm_cols(w):
    s = w.shape[:-1]
    return w.reshape(*s, HEADS, D_HEAD).swapaxes(-1, -2).reshape(*s, D_OUT)


def _node_proj_body(nf, wq, bq, wk, bk, wv, bv, q, k, v):
    x = nf[...]
    q[...] = jnp.dot(x, wq[...], preferred_element_type=jnp.float32) + bq[...]
    k[...] = jnp.dot(x, wk[...], preferred_element_type=jnp.float32) + bk[...]
    v[...] = jnp.dot(x, wv[...], preferred_element_type=jnp.float32) + bv[...]


def _node_proj(nf, wq, bq, wk, bk, wv, bv):
    n = nf.shape[0]
    grid = (n // NODE_BLK,)
    row = pl.BlockSpec((NODE_BLK, D_OUT), lambda i: (i, 0))
    full = pl.BlockSpec((D_OUT, D_OUT), lambda i: (0, 0))
    bias = pl.BlockSpec((1, D_OUT), lambda i: (0, 0))
    out = jax.ShapeDtypeStruct((n, D_OUT), jnp.float32)
    return pl.pallas_call(
        _node_proj_body,
        grid=grid,
        in_specs=[row, full, bias, full, bias, full, bias],
        out_specs=[row, row, row],
        out_shape=[out, out, out],
    )(nf, wq, bq, wk, bk, wv, bv)


def _edge_proj_body(t, ef, wt, bt, wet, wef, e):
    tf = jnp.cos(t[...] * wt[...] + bt[...])
    e[...] = (jnp.dot(tf, wet[...], preferred_element_type=jnp.float32)
              + jnp.dot(ef[...], wef[...], preferred_element_type=jnp.float32))


def _edge_proj(t2, ef, wt, bt, wet, wef):
    grid = (E_TOT // EDGE_BLK,)
    return pl.pallas_call(
        _edge_proj_body,
        grid=grid,
        in_specs=[
            pl.BlockSpec((EDGE_BLK, 1), lambda i: (i, 0)),
            pl.BlockSpec((EDGE_BLK, D_EDGE), lambda i: (i, 0)),
            pl.BlockSpec((1, T_DIM), lambda i: (0, 0)),
            pl.BlockSpec((1, T_DIM), lambda i: (0, 0)),
            pl.BlockSpec((T_DIM, D_OUT), lambda i: (0, 0)),
            pl.BlockSpec((D_EDGE, D_OUT), lambda i: (0, 0)),
        ],
        out_specs=pl.BlockSpec((EDGE_BLK, D_OUT), lambda i: (i, 0)),
        out_shape=jax.ShapeDtypeStruct((E_TOT, D_OUT), jnp.float32),
    )(t2, ef, wt, bt, wet, wef)


def _sc_edge_body(q_hbm, k_hbm, v_hbm, e_hbm, src_hbm, dst_hbm, zrow_hbm,
                  zden_hbm, agg_hbm, den_hbm,
                  src_v, dst_v, qb, kb, vb, eb, mb, wb, agg_sh, den_sh, sem):
    c = lax.axis_index("c")
    s = lax.axis_index("s")
    wid = c * NS + s

    r0 = s * RPS
    pltpu.sync_copy(zrow_hbm.at[pl.ds(r0, RPS)], agg_sh.at[pl.ds(r0, RPS)])
    pltpu.sync_copy(zden_hbm.at[pl.ds(r0, RPS)], den_sh.at[pl.ds(r0, RPS)])
    plsc.subcore_barrier()

    base = wid * EPW
    rot8 = (lax.iota(jnp.int32, (16,)) + 8) & 15

    def edge_body(b, carry):
        acc = jnp.zeros((16,), jnp.float32)
        for j in range(HEADS):
            sl = pl.ds(16 * j, 16)
            acc = acc + qb[b, sl] * (kb[b, sl] + eb[b, sl])
        a = acc + jnp.take(acc, rot8, mode="promise_in_bounds")
        w = jnp.exp(a * 0.25)
        wb[b, :] = w
        for j in range(HEADS):
            sl = pl.ds(16 * j, 16)
            mb[b, sl] = (vb[b, sl] + eb[b, sl]) * w
        return carry

    def batch_body(i, carry):
        off = base + i * EB
        pltpu.sync_copy(src_hbm.at[pl.ds(off, EB)], src_v)
        pltpu.sync_copy(dst_hbm.at[pl.ds(off, EB)], dst_v)
        cp_e = pltpu.async_copy(e_hbm.at[pl.ds(off, EB)], eb, sem)
        cp_q = pltpu.async_copy(q_hbm.at[dst_v], qb, sem)
        cp_k = pltpu.async_copy(k_hbm.at[src_v], kb, sem)
        cp_v = pltpu.async_copy(v_hbm.at[src_v], vb, sem)
        cp_e.wait()
        cp_q.wait()
        cp_k.wait()
        cp_v.wait()
        lax.fori_loop(0, EB, edge_body, 0)
        pltpu.sync_copy(mb, agg_sh.at[dst_v], add=True)
        pltpu.sync_copy(wb, den_sh.at[dst_v], add=True)
        return carry

    lax.fori_loop(0, NBATCH, batch_body, 0)

    plsc.subcore_barrier()
    pltpu.sync_copy(agg_sh.at[pl.ds(r0, RPS)], agg_hbm.at[c, pl.ds(r0, RPS)])
    pltpu.sync_copy(den_sh.at[pl.ds(r0, RPS)], den_hbm.at[c, pl.ds(r0, RPS)])


def _sc_edge_pass(q, k, v, e, src, dst):
    mesh = plsc.VectorSubcoreMesh(core_axis_name="c", subcore_axis_name="s")
    zrow = jnp.zeros((N_NODES, D_OUT), jnp.float32)
    zden = jnp.zeros((N_NODES, 16), jnp.float32)
    call = pl.kernel(
        _sc_edge_body,
        out_type=[
            jax.ShapeDtypeStruct((NC, N_NODES, D_OUT), jnp.float32),
            jax.ShapeDtypeStruct((NC, N_NODES, 16), jnp.float32),
        ],
        mesh=mesh,
        scratch_types=[
            pltpu.VMEM((EB,), jnp.int32),
            pltpu.VMEM((EB,), jnp.int32),
            pltpu.VMEM((EB, D_OUT), jnp.float32),
            pltpu.VMEM((EB, D_OUT), jnp.float32),
            pltpu.VMEM((EB, D_OUT), jnp.float32),
            pltpu.VMEM((EB, D_OUT), jnp.float32),
            pltpu.VMEM((EB, D_OUT), jnp.float32),
            pltpu.VMEM((EB, 16), jnp.float32),
            pltpu.VMEM_SHARED((N_NODES, D_OUT), jnp.float32),
            pltpu.VMEM_SHARED((N_NODES, 16), jnp.float32),
            pltpu.SemaphoreType.DMA,
        ],
    )
    return call(q, k, v, e, src, dst, zrow, zden)


def _final_body(a0, a1, d0, d1, nf, ws, bs, tile, out):
    den = d0[...] + d1[...] + 1e-16
    dent = jnp.dot(den, tile[...], preferred_element_type=jnp.float32,
                   precision=lax.Precision.HIGHEST)
    skip = jnp.dot(nf[...], ws[...], preferred_element_type=jnp.float32) + bs[...]
    out[...] = (a0[...] + a1[...]) / dent + skip


def _finalize(a0, a1, d0, d1, nf, ws, bs):
    tile = jnp.concatenate([jnp.eye(16, dtype=jnp.float32)] * HEADS, axis=1)
    grid = (N_NODES // NODE_BLK,)
    row = pl.BlockSpec((NODE_BLK, D_OUT), lambda i: (i, 0))
    den = pl.BlockSpec((NODE_BLK, 16), lambda i: (i, 0))
    return pl.pallas_call(
        _final_body,
        grid=grid,
        in_specs=[row, row, den, den, row,
                  pl.BlockSpec((D_OUT, D_OUT), lambda i: (0, 0)),
                  pl.BlockSpec((1, D_OUT), lambda i: (0, 0)),
                  pl.BlockSpec((16, D_OUT), lambda i: (0, 0))],
        out_specs=row,
        out_shape=jax.ShapeDtypeStruct((N_NODES, D_OUT), jnp.float32),
    )(a0, a1, d0, d1, nf, ws, bs, tile)


def kernel(edge_tuples, edge_feats, edge_times_rel, node_feats, w_time, b_time,
           W_q, b_q, W_k, b_k, W_v, b_v, W_e, W_skip, b_skip):
    src = edge_tuples[0]
    dst = edge_tuples[1]

    wq = _perm_cols(W_q)
    wk = _perm_cols(W_k)
    wv = _perm_cols(W_v)
    we = _perm_cols(W_e)
    wsk = _perm_cols(W_skip)
    bq = _perm_cols(b_q.reshape(1, D_OUT))
    bk = _perm_cols(b_k.reshape(1, D_OUT))
    bv = _perm_cols(b_v.reshape(1, D_OUT))
    bsk = _perm_cols(b_skip.reshape(1, D_OUT))

    q, k, v = _node_proj(node_feats, wq, bq, wk, bk, wv, bv)
    e = _edge_proj(edge_times_rel.reshape(E_TOT, 1), edge_feats,
                   w_time, b_time.reshape(1, T_DIM), we[:T_DIM], we[T_DIM:])
    agg, den = _sc_edge_pass(q, k, v, e, src, dst)
    outp = _finalize(agg[0], agg[1], den[0], den[1], node_feats, wsk, bsk)
    return outp.reshape(N_NODES, D_HEAD, HEADS).swapaxes(1, 2).reshape(N_NODES, D_OUT)


# speedup vs baseline: 14.5333x; 14.5333x over previous
import jax
import jax.numpy as jnp
from jax import lax
from jax.experimental import pallas as pl
from jax.experimental.pallas import tpu as pltpu
from jax.experimental.pallas import tpu_sc as plsc

N_NODES = 10000
E_TOT = 320000
HEADS = 8
D_HEAD = 16
D_OUT = 128
T_DIM = 32
D_EDGE = 16
DH = 64

NC = 2
NS = 16
NW = NC * NS
EPT = E_TOT // NS
EB = 80
NBATCH = EPT // EB
RPS = 624
REM = N_NODES - NS * RPS
N4 = N_NODES * 4

NODE_BLK = 400
EDGE_BLK = 2000


def _perm_cols(w):
    s = w.shape[:-1]
    w4 = w.reshape(*s, 2, 4, D_HEAD)
    w4 = jnp.swapaxes(w4, -1, -2)
    return w4.reshape(*s, D_OUT)


def _split_cores(w):
    k = w.shape[0]
    return w.reshape(k, NC, DH).swapaxes(0, 1)


def _node_proj_body(nf, wq, bq, wk, bk, wv, bv, q, k, v):
    x = nf[...]
    q[...] = jnp.dot(x, wq[0], preferred_element_type=jnp.float32) + bq[0]
    k[...] = jnp.dot(x, wk[0], preferred_element_type=jnp.float32) + bk[0]
    v[...] = jnp.dot(x, wv[0], preferred_element_type=jnp.float32) + bv[0]


def _node_proj(nf, wq, bq, wk, bk, wv, bv):
    grid = (NC, N_NODES // NODE_BLK)
    xrow = pl.BlockSpec((NODE_BLK, D_OUT), lambda c, i: (i, 0))
    wspec = pl.BlockSpec((1, D_OUT, DH), lambda c, i: (c, 0, 0))
    bspec = pl.BlockSpec((1, 1, DH), lambda c, i: (c, 0, 0))
    orow = pl.BlockSpec((NODE_BLK, DH),
                        lambda c, i: (c * (N_NODES // NODE_BLK) + i, 0))
    out = jax.ShapeDtypeStruct((NC * N_NODES, DH), jnp.float32)
    return pl.pallas_call(
        _node_proj_body,
        grid=grid,
        in_specs=[xrow, wspec, bspec, wspec, bspec, wspec, bspec],
        out_specs=[orow, orow, orow],
        out_shape=[out, out, out],
    )(nf, wq, bq, wk, bk, wv, bv)


def _edge_proj_body(t, ef, wt, bt, wet, wef, e):
    tf = jnp.cos(t[...] * wt[...] + bt[...])
    e[...] = (jnp.dot(tf, wet[0], preferred_element_type=jnp.float32)
              + jnp.dot(ef[...], wef[0], preferred_element_type=jnp.float32))


def _edge_proj(t2, ef, wt, bt, wet, wef):
    grid = (NC, E_TOT // EDGE_BLK)
    return pl.pallas_call(
        _edge_proj_body,
        grid=grid,
        in_specs=[
            pl.BlockSpec((EDGE_BLK, 1), lambda c, i: (i, 0)),
            pl.BlockSpec((EDGE_BLK, D_EDGE), lambda c, i: (i, 0)),
            pl.BlockSpec((1, T_DIM), lambda c, i: (0, 0)),
            pl.BlockSpec((1, T_DIM), lambda c, i: (0, 0)),
            pl.BlockSpec((1, T_DIM, DH), lambda c, i: (c, 0, 0)),
            pl.BlockSpec((1, D_EDGE, DH), lambda c, i: (c, 0, 0)),
        ],
        out_specs=pl.BlockSpec((EDGE_BLK, DH),
                               lambda c, i: (c * (E_TOT // EDGE_BLK) + i, 0)),
        out_shape=jax.ShapeDtypeStruct((NC * E_TOT, DH), jnp.float32),
    )(t2, ef, wt, bt, wet, wef)


def _sc_edge_body(q_hbm, k_hbm, v_hbm, e_hbm, src_hbm, dst_hbm,
                  agg_hbm, den_hbm,
                  src_v, dst_v, dstt_v, qb, kb, vb, eb, den_t, agg_sh, sem):
    c = lax.axis_index("c")
    s = lax.axis_index("s")
    lane16 = lax.iota(jnp.int32, 16)
    zero16 = jnp.zeros((16,), jnp.float32)

    def zden_body(i, carry):
        den_t[pl.ds(i * 16, 16)] = zero16
        return carry

    lax.fori_loop(0, (N4 + 16) // 16, zden_body, 0)

    def zkb_body(i, carry):
        for jj in range(DH // 16):
            kb[i, pl.ds(16 * jj, 16)] = zero16
        return carry

    lax.fori_loop(0, EB, zkb_body, 0)

    r0 = s * RPS
    rtail = NS * RPS
    for i in range(7):
        pltpu.sync_copy(kb.at[pl.ds(0, EB)], agg_sh.at[pl.ds(r0 + i * EB, EB)])
    pltpu.sync_copy(kb.at[pl.ds(0, RPS - 7 * EB)],
                    agg_sh.at[pl.ds(r0 + 7 * EB, RPS - 7 * EB)])

    @pl.when(s == NS - 1)
    def _zero_tail():
        pltpu.sync_copy(kb.at[pl.ds(0, REM)], agg_sh.at[pl.ds(rtail, REM)])

    plsc.subcore_barrier()

    base = s * EPT
    ebase = c * E_TOT
    nbase = c * N_NODES
    rot8 = (lane16 + 8) & 15
    rot4 = (lane16 + 4) & 15
    low4 = lane16 < 4

    def group_body(g, carry):
        dvec = dst_v[pl.ds(g * 16, 16)]
        for l in range(16):
            b = g * 16 + l
            acc = jnp.zeros((16,), jnp.float32)
            for j in range(DH // 16):
                sl = pl.ds(16 * j, 16)
                acc = acc + qb[b, sl] * (kb[b, sl] + eb[b, sl])
            a = acc + lax.gather(
                acc, rot8.reshape(16, 1),
                dimension_numbers=lax.GatherDimensionNumbers(
                    offset_dims=(), collapsed_slice_dims=(0,),
                    start_index_map=(0,)),
                slice_sizes=(1,), mode=lax.GatherScatterMode.PROMISE_IN_BOUNDS)
            a = a + lax.gather(
                a, rot4.reshape(16, 1),
                dimension_numbers=lax.GatherDimensionNumbers(
                    offset_dims=(), collapsed_slice_dims=(0,),
                    start_index_map=(0,)),
                slice_sizes=(1,), mode=lax.GatherScatterMode.PROMISE_IN_BOUNDS)
            w = jnp.exp(a * 0.25)
            for j in range(DH // 16):
                sl = pl.ds(16 * j, 16)
                vb[b, sl] = (vb[b, sl] + eb[b, sl]) * w
            f = jnp.where(low4, dvec[l] * 4 + lane16, N4 + lane16)
            plsc.addupdate_scatter(den_t, [f], w)
        return carry

    def idxoff_body(g, carry):
        sl = pl.ds(g * 16, 16)
        src_v[sl] = src_v[sl] + nbase
        dstt_v[sl] = dst_v[sl] + nbase
        return carry

    def batch_body(i, carry):
        off = base + i * EB
        pltpu.sync_copy(src_hbm.at[pl.ds(off, EB)], src_v)
        pltpu.sync_copy(dst_hbm.at[pl.ds(off, EB)], dst_v)
        lax.fori_loop(0, EB // 16, idxoff_body, 0)
        cp_e = pltpu.async_copy(e_hbm.at[pl.ds(ebase + off, EB)], eb, sem)
        cp_q = pltpu.async_copy(q_hbm.at[dstt_v], qb, sem)
        cp_k = pltpu.async_copy(k_hbm.at[src_v], kb, sem)
        cp_v = pltpu.async_copy(v_hbm.at[src_v], vb, sem)
        cp_e.wait()
        cp_q.wait()
        cp_k.wait()
        cp_v.wait()
        lax.fori_loop(0, EB // 16, group_body, 0)
        pltpu.sync_copy(vb, agg_sh.at[dst_v], add=True)
        return carry

    lax.fori_loop(0, NBATCH, batch_body, 0)

    plsc.subcore_barrier()
    pltpu.sync_copy(agg_sh.at[pl.ds(r0, RPS)], agg_hbm.at[c, pl.ds(r0, RPS)])

    @pl.when(s == NS - 1)
    def _drain_tail():
        pltpu.sync_copy(agg_sh.at[pl.ds(rtail, REM)], agg_hbm.at[c, pl.ds(rtail, REM)])

    pltpu.sync_copy(den_t.at[pl.ds(0, N4)], den_hbm.at[c, s])


def _sc_edge_pass(q, k, v, e, src, dst):
    mesh = plsc.VectorSubcoreMesh(core_axis_name="c", subcore_axis_name="s")
    call = pl.kernel(
        _sc_edge_body,
        out_type=[
            jax.ShapeDtypeStruct((NC, N_NODES, DH), jnp.float32),
            jax.ShapeDtypeStruct((NC, NS, N4), jnp.float32),
        ],
        mesh=mesh,
        compiler_params=pltpu.CompilerParams(needs_layout_passes=False,
                                            use_tc_tiling_on_sc=False),
        scratch_types=[
            pltpu.VMEM((EB,), jnp.int32),
            pltpu.VMEM((EB,), jnp.int32),
            pltpu.VMEM((EB,), jnp.int32),
            pltpu.VMEM((EB, DH), jnp.float32),
            pltpu.VMEM((EB, DH), jnp.float32),
            pltpu.VMEM((EB, DH), jnp.float32),
            pltpu.VMEM((EB, DH), jnp.float32),
            pltpu.VMEM((N4 + 16,), jnp.float32),
            pltpu.VMEM_SHARED((N_NODES, DH), jnp.float32),
            pltpu.SemaphoreType.DMA,
        ],
    )
    return call(q, k, v, e, src, dst)


def _final_body(a0, a1, d0, d1, nf, ws, bs, tile, out):
    den0 = jnp.sum(d0[...], axis=0) + 1e-16
    den1 = jnp.sum(d1[...], axis=0) + 1e-16
    dent0 = jnp.dot(den0, tile[...], preferred_element_type=jnp.float32,
                    precision=lax.Precision.HIGHEST)
    dent1 = jnp.dot(den1, tile[...], preferred_element_type=jnp.float32,
                    precision=lax.Precision.HIGHEST)
    agg = jnp.concatenate([a0[...] / dent0, a1[...] / dent1], axis=1)
    skip = jnp.dot(nf[...], ws[...], preferred_element_type=jnp.float32) + bs[...]
    out[...] = agg + skip


def _finalize(a0, a1, d0, d1, nf, ws, bs):
    tile = jnp.concatenate([jnp.eye(4, dtype=jnp.float32)] * (DH // 4), axis=1)
    grid = (N_NODES // NODE_BLK,)
    row = pl.BlockSpec((NODE_BLK, D_OUT), lambda i: (i, 0))
    arow = pl.BlockSpec((NODE_BLK, DH), lambda i: (i, 0))
    dspec = pl.BlockSpec((NS, NODE_BLK, 4), lambda i: (0, i, 0))
    return pl.pallas_call(
        _final_body,
        grid=grid,
        in_specs=[arow, arow, dspec, dspec, row,
                  pl.BlockSpec((D_OUT, D_OUT), lambda i: (0, 0)),
                  pl.BlockSpec((1, D_OUT), lambda i: (0, 0)),
                  pl.BlockSpec((4, DH), lambda i: (0, 0))],
        out_specs=row,
        out_shape=jax.ShapeDtypeStruct((N_NODES, D_OUT), jnp.float32),
    )(a0, a1, d0, d1, nf, ws, bs, tile)


def kernel(edge_tuples, edge_feats, edge_times_rel, node_feats, w_time, b_time,
           W_q, b_q, W_k, b_k, W_v, b_v, W_e, W_skip, b_skip):
    src = edge_tuples[0]
    dst = edge_tuples[1]

    wq = _split_cores(_perm_cols(W_q))
    wk = _split_cores(_perm_cols(W_k))
    wv = _split_cores(_perm_cols(W_v))
    we = _split_cores(_perm_cols(W_e))
    wsk = _perm_cols(W_skip)
    bq = _split_cores(_perm_cols(b_q.reshape(1, D_OUT)))
    bk = _split_cores(_perm_cols(b_k.reshape(1, D_OUT)))
    bv = _split_cores(_perm_cols(b_v.reshape(1, D_OUT)))
    bsk = _perm_cols(b_skip.reshape(1, D_OUT))

    q, k, v = _node_proj(node_feats, wq, bq, wk, bk, wv, bv)
    e = _edge_proj(edge_times_rel.reshape(E_TOT, 1), edge_feats,
                   w_time, b_time.reshape(1, T_DIM),
                   we[:, :T_DIM], we[:, T_DIM:])
    agg, den = _sc_edge_pass(q, k, v, e, src, dst)
    den4 = den.reshape(NC, NS, N_NODES, 4)
    outp = _finalize(agg[0], agg[1], den4[0], den4[1], node_feats, wsk, bsk)
    return (outp.reshape(N_NODES, 2, D_HEAD, 4).transpose(0, 1, 3, 2)
            .reshape(N_NODES, D_OUT))


# speedup vs baseline: 14.7638x; 1.0159x over previous
import jax
import jax.numpy as jnp
from jax import lax
from jax.experimental import pallas as pl
from jax.experimental.pallas import tpu as pltpu
from jax.experimental.pallas import tpu_sc as plsc

N_NODES = 10000
E_TOT = 320000
HEADS = 8
D_HEAD = 16
D_OUT = 128
T_DIM = 32
D_EDGE = 16
DH = 64

NC = 2
NS = 16
NW = NC * NS
EPT = E_TOT // NS
EB = 80
NBATCH = EPT // EB
RPS = 624
REM = N_NODES - NS * RPS
N4 = N_NODES * 4

NODE_BLK = 400
EDGE_BLK = 2000


def _perm_cols(w):
    s = w.shape[:-1]
    w4 = w.reshape(*s, 2, 4, D_HEAD)
    w4 = jnp.swapaxes(w4, -1, -2)
    return w4.reshape(*s, D_OUT)


def _split_cores(w):
    k = w.shape[0]
    return w.reshape(k, NC, DH).swapaxes(0, 1)


def _node_proj_body(nf, wq, bq, wk, bk, wv, bv, q, k, v):
    x = nf[...]
    q[...] = jnp.dot(x, wq[0], preferred_element_type=jnp.float32) + bq[0]
    k[...] = jnp.dot(x, wk[0], preferred_element_type=jnp.float32) + bk[0]
    v[...] = jnp.dot(x, wv[0], preferred_element_type=jnp.float32) + bv[0]


def _node_proj(nf, wq, bq, wk, bk, wv, bv):
    grid = (NC, N_NODES // NODE_BLK)
    xrow = pl.BlockSpec((NODE_BLK, D_OUT), lambda c, i: (i, 0))
    wspec = pl.BlockSpec((1, D_OUT, DH), lambda c, i: (c, 0, 0))
    bspec = pl.BlockSpec((1, 1, DH), lambda c, i: (c, 0, 0))
    orow = pl.BlockSpec((NODE_BLK, DH),
                        lambda c, i: (c * (N_NODES // NODE_BLK) + i, 0))
    out = jax.ShapeDtypeStruct((NC * N_NODES, DH), jnp.float32)
    return pl.pallas_call(
        _node_proj_body,
        grid=grid,
        in_specs=[xrow, wspec, bspec, wspec, bspec, wspec, bspec],
        out_specs=[orow, orow, orow],
        out_shape=[out, out, out],
    )(nf, wq, bq, wk, bk, wv, bv)


def _edge_proj_body(t, ef, wt, bt, wet, wef, e):
    tf = jnp.cos(t[...] * wt[...] + bt[...])
    e[...] = (jnp.dot(tf, wet[0], preferred_element_type=jnp.float32)
              + jnp.dot(ef[...], wef[0], preferred_element_type=jnp.float32))


def _edge_proj(t2, ef, wt, bt, wet, wef):
    grid = (NC, E_TOT // EDGE_BLK)
    return pl.pallas_call(
        _edge_proj_body,
        grid=grid,
        in_specs=[
            pl.BlockSpec((EDGE_BLK, 1), lambda c, i: (i, 0)),
            pl.BlockSpec((EDGE_BLK, D_EDGE), lambda c, i: (i, 0)),
            pl.BlockSpec((1, T_DIM), lambda c, i: (0, 0)),
            pl.BlockSpec((1, T_DIM), lambda c, i: (0, 0)),
            pl.BlockSpec((1, T_DIM, DH), lambda c, i: (c, 0, 0)),
            pl.BlockSpec((1, D_EDGE, DH), lambda c, i: (c, 0, 0)),
        ],
        out_specs=pl.BlockSpec((EDGE_BLK, DH),
                               lambda c, i: (c * (E_TOT // EDGE_BLK) + i, 0)),
        out_shape=jax.ShapeDtypeStruct((NC * E_TOT, DH), jnp.float32),
    )(t2, ef, wt, bt, wet, wef)


def _sc_edge_body(q_hbm, k_hbm, v_hbm, e_hbm, src_hbm, dst_hbm,
                  agg_hbm, den_hbm,
                  src_v, dst_v, dstt_v, qb, kb, vb, eb, den_t, agg_sh, sem):
    c = lax.axis_index("c")
    s = lax.axis_index("s")
    lane16 = lax.iota(jnp.int32, 16)
    zero16 = jnp.zeros((16,), jnp.float32)

    def zden_body(i, carry):
        den_t[pl.ds(i * 16, 16)] = zero16
        return carry

    lax.fori_loop(0, (N4 + 16) // 16, zden_body, 0)

    def zkb_body(i, carry):
        for jj in range(DH // 16):
            kb[i, pl.ds(16 * jj, 16)] = zero16
        return carry

    lax.fori_loop(0, EB, zkb_body, 0)

    r0 = s * RPS
    rtail = NS * RPS
    for i in range(7):
        pltpu.sync_copy(kb.at[pl.ds(0, EB)], agg_sh.at[pl.ds(r0 + i * EB, EB)])
    pltpu.sync_copy(kb.at[pl.ds(0, RPS - 7 * EB)],
                    agg_sh.at[pl.ds(r0 + 7 * EB, RPS - 7 * EB)])

    @pl.when(s == NS - 1)
    def _zero_tail():
        pltpu.sync_copy(kb.at[pl.ds(0, REM)], agg_sh.at[pl.ds(rtail, REM)])

    plsc.subcore_barrier()

    base = s * EPT
    ebase = c * E_TOT
    nbase = c * N_NODES
    rot8 = (lane16 + 8) & 15
    rot4 = (lane16 + 4) & 15
    low4 = lane16 < 4

    def group_body(g, carry):
        dvec = dst_v[pl.ds(g * 16, 16)]
        for l in range(16):
            b = g * 16 + l
            acc = jnp.zeros((16,), jnp.float32)
            for j in range(DH // 16):
                sl = pl.ds(16 * j, 16)
                acc = acc + qb[b, sl] * (kb[b, sl] + eb[b, sl])
            a = acc + lax.gather(
                acc, rot8.reshape(16, 1),
                dimension_numbers=lax.GatherDimensionNumbers(
                    offset_dims=(), collapsed_slice_dims=(0,),
                    start_index_map=(0,)),
                slice_sizes=(1,), mode=lax.GatherScatterMode.PROMISE_IN_BOUNDS)
            a = a + lax.gather(
                a, rot4.reshape(16, 1),
                dimension_numbers=lax.GatherDimensionNumbers(
                    offset_dims=(), collapsed_slice_dims=(0,),
                    start_index_map=(0,)),
                slice_sizes=(1,), mode=lax.GatherScatterMode.PROMISE_IN_BOUNDS)
            w = jnp.exp(a * 0.25)
            for j in range(DH // 16):
                sl = pl.ds(16 * j, 16)
                vb[b, sl] = (vb[b, sl] + eb[b, sl]) * w
            f = jnp.where(low4, dvec[l] * 4 + lane16, N4 + lane16)
            plsc.addupdate_scatter(den_t, [f], w)
        return carry

    def idxoff_body(g, carry):
        sl = pl.ds(g * 16, 16)
        src_v[sl] = src_v[sl] + nbase
        dstt_v[sl] = dst_v[sl] + nbase
        return carry

    def batch_body(i, carry):
        off = base + i * EB
        pltpu.sync_copy(src_hbm.at[pl.ds(off, EB)], src_v)
        pltpu.sync_copy(dst_hbm.at[pl.ds(off, EB)], dst_v)
        lax.fori_loop(0, EB // 16, idxoff_body, 0)
        cp_e = pltpu.async_copy(e_hbm.at[pl.ds(ebase + off, EB)], eb, sem)
        cp_q = pltpu.async_copy(q_hbm.at[dstt_v], qb, sem)
        cp_k = pltpu.async_copy(k_hbm.at[src_v], kb, sem)
        cp_v = pltpu.async_copy(v_hbm.at[src_v], vb, sem)
        cp_e.wait()
        cp_q.wait()
        cp_k.wait()
        cp_v.wait()
        lax.fori_loop(0, EB // 16, group_body, 0)
        pltpu.sync_copy(vb, agg_sh.at[dst_v], add=True)
        return carry

    lax.fori_loop(0, NBATCH, batch_body, 0)

    plsc.subcore_barrier()
    pltpu.sync_copy(agg_sh.at[pl.ds(r0, RPS)], agg_hbm.at[c, pl.ds(r0, RPS)])

    @pl.when(s == NS - 1)
    def _drain_tail():
        pltpu.sync_copy(agg_sh.at[pl.ds(rtail, REM)], agg_hbm.at[c, pl.ds(rtail, REM)])

    pltpu.sync_copy(den_t.at[pl.ds(0, N4)], den_hbm.at[c, s])


def _sc_edge_pass(q, k, v, e, src, dst):
    mesh = plsc.VectorSubcoreMesh(core_axis_name="c", subcore_axis_name="s")
    call = pl.kernel(
        _sc_edge_body,
        out_type=[
            jax.ShapeDtypeStruct((NC, N_NODES, DH), jnp.float32),
            jax.ShapeDtypeStruct((NC, NS, N4), jnp.float32),
        ],
        mesh=mesh,
        compiler_params=pltpu.CompilerParams(needs_layout_passes=False,
                                            use_tc_tiling_on_sc=False),
        scratch_types=[
            pltpu.VMEM((EB,), jnp.int32),
            pltpu.VMEM((EB,), jnp.int32),
            pltpu.VMEM((EB,), jnp.int32),
            pltpu.VMEM((EB, DH), jnp.float32),
            pltpu.VMEM((EB, DH), jnp.float32),
            pltpu.VMEM((EB, DH), jnp.float32),
            pltpu.VMEM((EB, DH), jnp.float32),
            pltpu.VMEM((N4 + 16,), jnp.float32),
            pltpu.VMEM_SHARED((N_NODES, DH), jnp.float32),
            pltpu.SemaphoreType.DMA,
        ],
    )
    return call(q, k, v, e, src, dst)


def _final_body(aref, dref, nf, ws, bs, tile, perm, out):
    den0 = jnp.sum(dref[0], axis=0) + 1e-16
    den1 = jnp.sum(dref[1], axis=0) + 1e-16
    dent0 = jnp.dot(den0, tile[...], preferred_element_type=jnp.float32,
                    precision=lax.Precision.HIGHEST)
    dent1 = jnp.dot(den1, tile[...], preferred_element_type=jnp.float32,
                    precision=lax.Precision.HIGHEST)
    agg = jnp.concatenate([aref[0] / dent0, aref[1] / dent1], axis=1)
    res = agg + jnp.dot(nf[...], ws[...], preferred_element_type=jnp.float32) + bs[...]
    out[...] = jnp.dot(res, perm[...], preferred_element_type=jnp.float32,
                       precision=lax.Precision.HIGHEST)


def _finalize(agg, den, nf, ws, bs):
    tile = jnp.concatenate([jnp.eye(4, dtype=jnp.float32)] * (DH // 4), axis=1)
    perm = _perm_cols(jnp.eye(D_OUT, dtype=jnp.float32).T).T
    grid = (N_NODES // NODE_BLK,)
    row = pl.BlockSpec((NODE_BLK, D_OUT), lambda i: (i, 0))
    aspec = pl.BlockSpec((NC, NODE_BLK, DH), lambda i: (0, i, 0))
    dspec = pl.BlockSpec((NC, NS, NODE_BLK, 4), lambda i: (0, 0, i, 0))
    return pl.pallas_call(
        _final_body,
        grid=grid,
        in_specs=[aspec, dspec, row,
                  pl.BlockSpec((D_OUT, D_OUT), lambda i: (0, 0)),
                  pl.BlockSpec((1, D_OUT), lambda i: (0, 0)),
                  pl.BlockSpec((4, DH), lambda i: (0, 0)),
                  pl.BlockSpec((D_OUT, D_OUT), lambda i: (0, 0))],
        out_specs=row,
        out_shape=jax.ShapeDtypeStruct((N_NODES, D_OUT), jnp.float32),
    )(agg, den, nf, ws, bs, tile, perm)


def kernel(edge_tuples, edge_feats, edge_times_rel, node_feats, w_time, b_time,
           W_q, b_q, W_k, b_k, W_v, b_v, W_e, W_skip, b_skip):
    src = edge_tuples[0]
    dst = edge_tuples[1]

    wq = _split_cores(_perm_cols(W_q))
    wk = _split_cores(_perm_cols(W_k))
    wv = _split_cores(_perm_cols(W_v))
    we = _split_cores(_perm_cols(W_e))
    wsk = _perm_cols(W_skip)
    bq = _split_cores(_perm_cols(b_q.reshape(1, D_OUT)))
    bk = _split_cores(_perm_cols(b_k.reshape(1, D_OUT)))
    bv = _split_cores(_perm_cols(b_v.reshape(1, D_OUT)))
    bsk = _perm_cols(b_skip.reshape(1, D_OUT))

    q, k, v = _node_proj(node_feats, wq, bq, wk, bk, wv, bv)
    e = _edge_proj(edge_times_rel.reshape(E_TOT, 1), edge_feats,
                   w_time, b_time.reshape(1, T_DIM),
                   we[:, :T_DIM], we[:, T_DIM:])
    agg, den = _sc_edge_pass(q, k, v, e, src, dst)
    den4 = den.reshape(NC, NS, N_NODES, 4)
    return _finalize(agg, den4, node_feats, wsk, bsk)


# speedup vs baseline: 20.5152x; 1.3896x over previous
import jax
import jax.numpy as jnp
from jax import lax
from jax.experimental import pallas as pl
from jax.experimental.pallas import tpu as pltpu
from jax.experimental.pallas import tpu_sc as plsc

N_NODES = 10000
E_TOT = 320000
HEADS = 8
D_HEAD = 16
D_OUT = 128
T_DIM = 32
D_EDGE = 16
DH = 64

NC = 2
NS = 16
NW = NC * NS
EPT = E_TOT // NS
EB = 80
NBATCH = EPT // EB
RPS = 624
REM = N_NODES - NS * RPS
N4 = N_NODES * 4

NODE_BLK = 400
EDGE_BLK = 2000


def _perm_cols(w):
    s = w.shape[:-1]
    w4 = w.reshape(*s, 2, 4, D_HEAD)
    w4 = jnp.swapaxes(w4, -1, -2)
    return w4.reshape(*s, D_OUT)


def _split_cores(w):
    k = w.shape[0]
    return w.reshape(k, NC, DH).swapaxes(0, 1)


def _node_proj_body(nf, wq, bq, wk, bk, wv, bv, q, k, v):
    x = nf[...]
    q[...] = jnp.dot(x, wq[0], preferred_element_type=jnp.float32) + bq[0]
    k[...] = jnp.dot(x, wk[0], preferred_element_type=jnp.float32) + bk[0]
    v[...] = jnp.dot(x, wv[0], preferred_element_type=jnp.float32) + bv[0]


def _node_proj(nf, wq, bq, wk, bk, wv, bv):
    grid = (NC, N_NODES // NODE_BLK)
    xrow = pl.BlockSpec((NODE_BLK, D_OUT), lambda c, i: (i, 0))
    wspec = pl.BlockSpec((1, D_OUT, DH), lambda c, i: (c, 0, 0))
    bspec = pl.BlockSpec((1, 1, DH), lambda c, i: (c, 0, 0))
    orow = pl.BlockSpec((NODE_BLK, DH),
                        lambda c, i: (c * (N_NODES // NODE_BLK) + i, 0))
    out = jax.ShapeDtypeStruct((NC * N_NODES, DH), jnp.float32)
    return pl.pallas_call(
        _node_proj_body,
        grid=grid,
        in_specs=[xrow, wspec, bspec, wspec, bspec, wspec, bspec],
        out_specs=[orow, orow, orow],
        out_shape=[out, out, out],
    )(nf, wq, bq, wk, bk, wv, bv)


_COSP = (0.9999999922898466, -19.73920555348366, 64.93917219630474,
         -85.45116501827795, 60.176223171395506, -26.000498057793394,
         6.575565933423059)


def _edge_proj_body(t, ef, w2, b2, wet, wef, e0, e1):
    u = t[...] * w2[...] + b2[...]
    u = u - jnp.round(u)
    z = u * u
    tf = jnp.float32(_COSP[6])
    for cc in _COSP[5::-1]:
        tf = tf * z + jnp.float32(cc)
    res = (jnp.dot(tf, wet[...], preferred_element_type=jnp.float32)
           + jnp.dot(ef[...], wef[...], preferred_element_type=jnp.float32))
    e0[...] = res[:, :DH]
    e1[...] = res[:, DH:]


def _edge_proj(t2, ef, w2, b2, wet, wef):
    grid = (E_TOT // EDGE_BLK,)
    out = jax.ShapeDtypeStruct((E_TOT, DH), jnp.float32)
    orow = pl.BlockSpec((EDGE_BLK, DH), lambda i: (i, 0))
    return pl.pallas_call(
        _edge_proj_body,
        grid=grid,
        in_specs=[
            pl.BlockSpec((EDGE_BLK, 1), lambda i: (i, 0)),
            pl.BlockSpec((EDGE_BLK, D_EDGE), lambda i: (i, 0)),
            pl.BlockSpec((1, T_DIM), lambda i: (0, 0)),
            pl.BlockSpec((1, T_DIM), lambda i: (0, 0)),
            pl.BlockSpec((T_DIM, D_OUT), lambda i: (0, 0)),
            pl.BlockSpec((D_EDGE, D_OUT), lambda i: (0, 0)),
        ],
        out_specs=[orow, orow],
        out_shape=[out, out],
    )(t2, ef, w2, b2, wet, wef)


def _sc_edge_body(q_hbm, k_hbm, v_hbm, e0_hbm, e1_hbm, src_hbm, dst_hbm,
                  agg_hbm, den_hbm,
                  src_v, dst_v, dstt_v, qb, kb, vb, eb, den_t, agg_sh, sem):
    c = lax.axis_index("c")
    s = lax.axis_index("s")
    lane16 = lax.iota(jnp.int32, 16)
    zero16 = jnp.zeros((16,), jnp.float32)

    def zden_body(i, carry):
        den_t[pl.ds(i * 16, 16)] = zero16
        return carry

    lax.fori_loop(0, (N4 + 16) // 16, zden_body, 0)

    def zkb_body(i, carry):
        for jj in range(DH // 16):
            kb[i, pl.ds(16 * jj, 16)] = zero16
        return carry

    lax.fori_loop(0, EB, zkb_body, 0)

    r0 = s * RPS
    rtail = NS * RPS
    for i in range(7):
        pltpu.sync_copy(kb.at[pl.ds(0, EB)], agg_sh.at[pl.ds(r0 + i * EB, EB)])
    pltpu.sync_copy(kb.at[pl.ds(0, RPS - 7 * EB)],
                    agg_sh.at[pl.ds(r0 + 7 * EB, RPS - 7 * EB)])

    @pl.when(s == NS - 1)
    def _zero_tail():
        pltpu.sync_copy(kb.at[pl.ds(0, REM)], agg_sh.at[pl.ds(rtail, REM)])

    plsc.subcore_barrier()

    base = s * EPT
    nbase = c * N_NODES
    rot8 = (lane16 + 8) & 15
    rot4 = (lane16 + 4) & 15
    low4 = lane16 < 4

    def group_body(g, carry):
        dvec = dst_v[pl.ds(g * 16, 16)]
        for l in range(16):
            b = g * 16 + l
            acc = jnp.zeros((16,), jnp.float32)
            for j in range(DH // 16):
                sl = pl.ds(16 * j, 16)
                acc = acc + qb[b, sl] * (kb[b, sl] + eb[b, sl])
            a = acc + lax.gather(
                acc, rot8.reshape(16, 1),
                dimension_numbers=lax.GatherDimensionNumbers(
                    offset_dims=(), collapsed_slice_dims=(0,),
                    start_index_map=(0,)),
                slice_sizes=(1,), mode=lax.GatherScatterMode.PROMISE_IN_BOUNDS)
            a = a + lax.gather(
                a, rot4.reshape(16, 1),
                dimension_numbers=lax.GatherDimensionNumbers(
                    offset_dims=(), collapsed_slice_dims=(0,),
                    start_index_map=(0,)),
                slice_sizes=(1,), mode=lax.GatherScatterMode.PROMISE_IN_BOUNDS)
            w = jnp.exp(a * 0.25)
            for j in range(DH // 16):
                sl = pl.ds(16 * j, 16)
                vb[b, sl] = (vb[b, sl] + eb[b, sl]) * w
            f = jnp.where(low4, dvec[l] * 4 + lane16, N4 + lane16)
            plsc.addupdate_scatter(den_t, [f], w)
        return carry

    def idxoff_body(g, carry):
        sl = pl.ds(g * 16, 16)
        src_v[sl] = src_v[sl] + nbase
        dstt_v[sl] = dst_v[sl] + nbase
        return carry

    def batch_body(i, carry):
        off = base + i * EB
        pltpu.sync_copy(src_hbm.at[pl.ds(off, EB)], src_v)
        pltpu.sync_copy(dst_hbm.at[pl.ds(off, EB)], dst_v)
        lax.fori_loop(0, EB // 16, idxoff_body, 0)
        cp_q = pltpu.async_copy(q_hbm.at[dstt_v], qb, sem)
        cp_k = pltpu.async_copy(k_hbm.at[src_v], kb, sem)
        cp_v = pltpu.async_copy(v_hbm.at[src_v], vb, sem)

        @pl.when(c == 0)
        def _load_e0():
            pltpu.sync_copy(e0_hbm.at[pl.ds(off, EB)], eb)

        @pl.when(c == 1)
        def _load_e1():
            pltpu.sync_copy(e1_hbm.at[pl.ds(off, EB)], eb)

        cp_q.wait()
        cp_k.wait()
        cp_v.wait()
        lax.fori_loop(0, EB // 16, group_body, 0)
        pltpu.sync_copy(vb, agg_sh.at[dst_v], add=True)
        return carry

    lax.fori_loop(0, NBATCH, batch_body, 0)

    plsc.subcore_barrier()
    pltpu.sync_copy(agg_sh.at[pl.ds(r0, RPS)], agg_hbm.at[c, pl.ds(r0, RPS)])

    @pl.when(s == NS - 1)
    def _drain_tail():
        pltpu.sync_copy(agg_sh.at[pl.ds(rtail, REM)], agg_hbm.at[c, pl.ds(rtail, REM)])

    pltpu.sync_copy(den_t.at[pl.ds(0, N4)], den_hbm.at[c, s])


def _sc_edge_pass(q, k, v, e0, e1, src, dst):
    mesh = plsc.VectorSubcoreMesh(core_axis_name="c", subcore_axis_name="s")
    call = pl.kernel(
        _sc_edge_body,
        out_type=[
            jax.ShapeDtypeStruct((NC, N_NODES, DH), jnp.float32),
            jax.ShapeDtypeStruct((NC, NS, N4), jnp.float32),
        ],
        mesh=mesh,
        compiler_params=pltpu.CompilerParams(needs_layout_passes=False,
                                            use_tc_tiling_on_sc=False),
        scratch_types=[
            pltpu.VMEM((EB,), jnp.int32),
            pltpu.VMEM((EB,), jnp.int32),
            pltpu.VMEM((EB,), jnp.int32),
            pltpu.VMEM((EB, DH), jnp.float32),
            pltpu.VMEM((EB, DH), jnp.float32),
            pltpu.VMEM((EB, DH), jnp.float32),
            pltpu.VMEM((EB, DH), jnp.float32),
            pltpu.VMEM((N4 + 16,), jnp.float32),
            pltpu.VMEM_SHARED((N_NODES, DH), jnp.float32),
            pltpu.SemaphoreType.DMA,
        ],
    )
    return call(q, k, v, e0, e1, src, dst)


def _final_body(aref, dref, nf, ws, bs, tile, perm, out):
    den0 = jnp.sum(dref[0], axis=0) + 1e-16
    den1 = jnp.sum(dref[1], axis=0) + 1e-16
    dent0 = jnp.dot(den0, tile[...], preferred_element_type=jnp.float32,
                    precision=lax.Precision.HIGHEST)
    dent1 = jnp.dot(den1, tile[...], preferred_element_type=jnp.float32,
                    precision=lax.Precision.HIGHEST)
    agg = jnp.concatenate([aref[0] / dent0, aref[1] / dent1], axis=1)
    res = agg + jnp.dot(nf[...], ws[...], preferred_element_type=jnp.float32) + bs[...]
    out[...] = jnp.dot(res, perm[...], preferred_element_type=jnp.float32,
                       precision=lax.Precision.HIGHEST)


def _finalize(agg, den, nf, ws, bs):
    tile = jnp.concatenate([jnp.eye(4, dtype=jnp.float32)] * (DH // 4), axis=1)
    perm = _perm_cols(jnp.eye(D_OUT, dtype=jnp.float32).T).T
    grid = (N_NODES // NODE_BLK,)
    row = pl.BlockSpec((NODE_BLK, D_OUT), lambda i: (i, 0))
    aspec = pl.BlockSpec((NC, NODE_BLK, DH), lambda i: (0, i, 0))
    dspec = pl.BlockSpec((NC, NS, NODE_BLK, 4), lambda i: (0, 0, i, 0))
    return pl.pallas_call(
        _final_body,
        grid=grid,
        in_specs=[aspec, dspec, row,
                  pl.BlockSpec((D_OUT, D_OUT), lambda i: (0, 0)),
                  pl.BlockSpec((1, D_OUT), lambda i: (0, 0)),
                  pl.BlockSpec((4, DH), lambda i: (0, 0)),
                  pl.BlockSpec((D_OUT, D_OUT), lambda i: (0, 0))],
        out_specs=row,
        out_shape=jax.ShapeDtypeStruct((N_NODES, D_OUT), jnp.float32),
    )(agg, den, nf, ws, bs, tile, perm)


def kernel(edge_tuples, edge_feats, edge_times_rel, node_feats, w_time, b_time,
           W_q, b_q, W_k, b_k, W_v, b_v, W_e, W_skip, b_skip):
    src = edge_tuples[0]
    dst = edge_tuples[1]

    wq = _split_cores(_perm_cols(W_q))
    wk = _split_cores(_perm_cols(W_k))
    wv = _split_cores(_perm_cols(W_v))
    we = _perm_cols(W_e)
    wsk = _perm_cols(W_skip)
    bq = _split_cores(_perm_cols(b_q.reshape(1, D_OUT)))
    bk = _split_cores(_perm_cols(b_k.reshape(1, D_OUT)))
    bv = _split_cores(_perm_cols(b_v.reshape(1, D_OUT)))
    bsk = _perm_cols(b_skip.reshape(1, D_OUT))

    q, k, v = _node_proj(node_feats, wq, bq, wk, bk, wv, bv)
    inv2pi = jnp.float32(1.0 / (2.0 * jnp.pi))
    e0, e1 = _edge_proj(edge_times_rel.reshape(E_TOT, 1), edge_feats,
                        w_time * inv2pi, b_time.reshape(1, T_DIM) * inv2pi,
                        we[:T_DIM], we[T_DIM:])
    agg, den = _sc_edge_pass(q, k, v, e0, e1, src, dst)
    den4 = den.reshape(NC, NS, N_NODES, 4)
    return _finalize(agg, den4, node_feats, wsk, bsk)


# speedup vs baseline: 23.9823x; 1.1690x over previous
import jax
import jax.numpy as jnp
from jax import lax
from jax.experimental import pallas as pl
from jax.experimental.pallas import tpu as pltpu
from jax.experimental.pallas import tpu_sc as plsc

N_NODES = 10000
E_TOT = 320000
HEADS = 8
D_HEAD = 16
D_OUT = 128
T_DIM = 32
D_EDGE = 16
DH = 64

NC = 2
NS = 16
NW = NC * NS
EPT = E_TOT // NS
EB = 80
NBATCH = EPT // EB
RPS = 624
REM = N_NODES - NS * RPS
N4 = N_NODES * 4

NODE_BLK = 400
EDGE_BLK = 2000


def _perm_cols(w):
    s = w.shape[:-1]
    w4 = w.reshape(*s, 2, 4, D_HEAD)
    w4 = jnp.swapaxes(w4, -1, -2)
    return w4.reshape(*s, D_OUT)


def _split_cores(w):
    k = w.shape[0]
    return w.reshape(k, NC, DH).swapaxes(0, 1)


def _node_proj_body(nf, wq, bq, wk, bk, wv, bv, q, k, v):
    x = nf[...]
    q[...] = jnp.dot(x, wq[0], preferred_element_type=jnp.float32) + bq[0]
    k[...] = jnp.dot(x, wk[0], preferred_element_type=jnp.float32) + bk[0]
    v[...] = jnp.dot(x, wv[0], preferred_element_type=jnp.float32) + bv[0]


def _node_proj(nf, wq, bq, wk, bk, wv, bv):
    grid = (NC, N_NODES // NODE_BLK)
    xrow = pl.BlockSpec((NODE_BLK, D_OUT), lambda c, i: (i, 0))
    wspec = pl.BlockSpec((1, D_OUT, DH), lambda c, i: (c, 0, 0))
    bspec = pl.BlockSpec((1, 1, DH), lambda c, i: (c, 0, 0))
    orow = pl.BlockSpec((NODE_BLK, DH),
                        lambda c, i: (c * (N_NODES // NODE_BLK) + i, 0))
    out = jax.ShapeDtypeStruct((NC * N_NODES, DH), jnp.float32)
    return pl.pallas_call(
        _node_proj_body,
        grid=grid,
        in_specs=[xrow, wspec, bspec, wspec, bspec, wspec, bspec],
        out_specs=[orow, orow, orow],
        out_shape=[out, out, out],
    )(nf, wq, bq, wk, bk, wv, bv)


_COSP = (0.9999999922898466, -19.73920555348366, 64.93917219630474,
         -85.45116501827795, 60.176223171395506, -26.000498057793394,
         6.575565933423059)


def _edge_proj_body(t, ef, w2, b2, wet, wef, e0, e1):
    u = t[...] * w2[...] + b2[...]
    u = u - jnp.round(u)
    z = u * u
    tf = jnp.float32(_COSP[6])
    for cc in _COSP[5::-1]:
        tf = tf * z + jnp.float32(cc)
    res = (jnp.dot(tf, wet[...], preferred_element_type=jnp.float32)
           + jnp.dot(ef[...], wef[...], preferred_element_type=jnp.float32))
    e0[...] = res[:, :DH]
    e1[...] = res[:, DH:]


def _edge_proj(t2, ef, w2, b2, wet, wef):
    grid = (E_TOT // EDGE_BLK,)
    out = jax.ShapeDtypeStruct((E_TOT, DH), jnp.float32)
    orow = pl.BlockSpec((EDGE_BLK, DH), lambda i: (i, 0))
    return pl.pallas_call(
        _edge_proj_body,
        grid=grid,
        in_specs=[
            pl.BlockSpec((EDGE_BLK, 1), lambda i: (i, 0)),
            pl.BlockSpec((EDGE_BLK, D_EDGE), lambda i: (i, 0)),
            pl.BlockSpec((1, T_DIM), lambda i: (0, 0)),
            pl.BlockSpec((1, T_DIM), lambda i: (0, 0)),
            pl.BlockSpec((T_DIM, D_OUT), lambda i: (0, 0)),
            pl.BlockSpec((D_EDGE, D_OUT), lambda i: (0, 0)),
        ],
        out_specs=[orow, orow],
        out_shape=[out, out],
    )(t2, ef, w2, b2, wet, wef)


def _sc_edge_body(q_hbm, k_hbm, v_hbm, e0_hbm, e1_hbm, src_hbm, dst_hbm,
                  agg_hbm, den_hbm,
                  src_v, dst_v, dstt_v, qb, kb, vb, eb,
                  src_v2, dst_v2, dstt_v2, qb2, kb2, vb2, eb2,
                  den_t, agg_sh, sem, sem2):
    c = lax.axis_index("c")
    s = lax.axis_index("s")
    lane16 = lax.iota(jnp.int32, 16)
    zero16 = jnp.zeros((16,), jnp.float32)

    def zden_body(i, carry):
        den_t[pl.ds(i * 16, 16)] = zero16
        return carry

    lax.fori_loop(0, (N4 + 16) // 16, zden_body, 0)

    def zkb_body(i, carry):
        for jj in range(DH // 16):
            kb[i, pl.ds(16 * jj, 16)] = zero16
        return carry

    lax.fori_loop(0, EB, zkb_body, 0)

    r0 = s * RPS
    rtail = NS * RPS
    for i in range(7):
        pltpu.sync_copy(kb.at[pl.ds(0, EB)], agg_sh.at[pl.ds(r0 + i * EB, EB)])
    pltpu.sync_copy(kb.at[pl.ds(0, RPS - 7 * EB)],
                    agg_sh.at[pl.ds(r0 + 7 * EB, RPS - 7 * EB)])

    @pl.when(s == NS - 1)
    def _zero_tail():
        pltpu.sync_copy(kb.at[pl.ds(0, REM)], agg_sh.at[pl.ds(rtail, REM)])

    plsc.subcore_barrier()

    base = s * EPT
    nbase = c * N_NODES
    rot8 = (lane16 + 8) & 15
    rot4 = (lane16 + 4) & 15
    low4 = lane16 < 4

    def make_group_body(qb, kb, vb, eb, dst_v):
        def group_body(g, carry):
            dvec = dst_v[pl.ds(g * 16, 16)]
            for l in range(16):
                b = g * 16 + l
                acc = jnp.zeros((16,), jnp.float32)
                for j in range(DH // 16):
                    sl = pl.ds(16 * j, 16)
                    acc = acc + qb[b, sl] * (kb[b, sl] + eb[b, sl])
                a = acc + lax.gather(
                    acc, rot8.reshape(16, 1),
                    dimension_numbers=lax.GatherDimensionNumbers(
                        offset_dims=(), collapsed_slice_dims=(0,),
                        start_index_map=(0,)),
                    slice_sizes=(1,),
                    mode=lax.GatherScatterMode.PROMISE_IN_BOUNDS)
                a = a + lax.gather(
                    a, rot4.reshape(16, 1),
                    dimension_numbers=lax.GatherDimensionNumbers(
                        offset_dims=(), collapsed_slice_dims=(0,),
                        start_index_map=(0,)),
                    slice_sizes=(1,),
                    mode=lax.GatherScatterMode.PROMISE_IN_BOUNDS)
                w = jnp.exp(a * 0.25)
                for j in range(DH // 16):
                    sl = pl.ds(16 * j, 16)
                    vb[b, sl] = (vb[b, sl] + eb[b, sl]) * w
                f = jnp.where(low4, dvec[l] * 4 + lane16, N4 + lane16)
                plsc.addupdate_scatter(den_t, [f], w)
            return carry
        return group_body

    bufsets = ((src_v, dst_v, dstt_v, qb, kb, vb, eb, sem),
               (src_v2, dst_v2, dstt_v2, qb2, kb2, vb2, eb2, sem2))

    def fire(i, bs):
        sv, dv, dtv, qx, kx, vx, ex, sm = bufsets[bs]
        off = base + i * EB
        pltpu.sync_copy(src_hbm.at[pl.ds(off, EB)], sv)
        pltpu.sync_copy(dst_hbm.at[pl.ds(off, EB)], dv)

        def idxoff_body(g, carry):
            sl = pl.ds(g * 16, 16)
            sv[sl] = sv[sl] + nbase
            dtv[sl] = dv[sl] + nbase
            return carry

        lax.fori_loop(0, EB // 16, idxoff_body, 0)
        pltpu.async_copy(q_hbm.at[dtv], qx, sm)
        pltpu.async_copy(k_hbm.at[sv], kx, sm)
        pltpu.async_copy(v_hbm.at[sv], vx, sm)

        @pl.when(c == 0)
        def _load_e0():
            pltpu.async_copy(e0_hbm.at[pl.ds(off, EB)], ex, sm)

        @pl.when(c == 1)
        def _load_e1():
            pltpu.async_copy(e1_hbm.at[pl.ds(off, EB)], ex, sm)

    def consume(i, bs):
        sv, dv, dtv, qx, kx, vx, ex, sm = bufsets[bs]
        pltpu.make_async_copy(q_hbm.at[dtv], qx, sm).wait()
        pltpu.make_async_copy(k_hbm.at[sv], kx, sm).wait()
        pltpu.make_async_copy(v_hbm.at[sv], vx, sm).wait()
        pltpu.make_async_copy(e0_hbm.at[pl.ds(0, EB)], ex, sm).wait()
        lax.fori_loop(0, EB // 16, make_group_body(qx, kx, vx, ex, dv), 0)
        pltpu.sync_copy(vx, agg_sh.at[dv], add=True)

    fire(0, 0)

    def pair_body(p, carry):
        fire(2 * p + 1, 1)
        consume(2 * p, 0)

        @pl.when(p + 1 < NBATCH // 2)
        def _fire_next():
            fire(2 * p + 2, 0)

        consume(2 * p + 1, 1)
        return carry

    lax.fori_loop(0, NBATCH // 2, pair_body, 0)

    plsc.subcore_barrier()
    pltpu.sync_copy(agg_sh.at[pl.ds(r0, RPS)], agg_hbm.at[c, pl.ds(r0, RPS)])

    @pl.when(s == NS - 1)
    def _drain_tail():
        pltpu.sync_copy(agg_sh.at[pl.ds(rtail, REM)], agg_hbm.at[c, pl.ds(rtail, REM)])

    pltpu.sync_copy(den_t.at[pl.ds(0, N4)], den_hbm.at[c, s])


def _sc_edge_pass(q, k, v, e0, e1, src, dst):
    mesh = plsc.VectorSubcoreMesh(core_axis_name="c", subcore_axis_name="s")
    call = pl.kernel(
        _sc_edge_body,
        out_type=[
            jax.ShapeDtypeStruct((NC, N_NODES, DH), jnp.float32),
            jax.ShapeDtypeStruct((NC, NS, N4), jnp.float32),
        ],
        mesh=mesh,
        compiler_params=pltpu.CompilerParams(needs_layout_passes=False,
                                            use_tc_tiling_on_sc=False),
        scratch_types=[
            pltpu.VMEM((EB,), jnp.int32),
            pltpu.VMEM((EB,), jnp.int32),
            pltpu.VMEM((EB,), jnp.int32),
            pltpu.VMEM((EB, DH), jnp.float32),
            pltpu.VMEM((EB, DH), jnp.float32),
            pltpu.VMEM((EB, DH), jnp.float32),
            pltpu.VMEM((EB, DH), jnp.float32),
            pltpu.VMEM((EB,), jnp.int32),
            pltpu.VMEM((EB,), jnp.int32),
            pltpu.VMEM((EB,), jnp.int32),
            pltpu.VMEM((EB, DH), jnp.float32),
            pltpu.VMEM((EB, DH), jnp.float32),
            pltpu.VMEM((EB, DH), jnp.float32),
            pltpu.VMEM((EB, DH), jnp.float32),
            pltpu.VMEM((N4 + 16,), jnp.float32),
            pltpu.VMEM_SHARED((N_NODES, DH), jnp.float32),
            pltpu.SemaphoreType.DMA,
            pltpu.SemaphoreType.DMA,
        ],
    )
    return call(q, k, v, e0, e1, src, dst)


def _final_body(aref, dref, nf, ws, bs, tile, perm, out):
    den0 = jnp.sum(dref[0], axis=0) + 1e-16
    den1 = jnp.sum(dref[1], axis=0) + 1e-16
    dent0 = jnp.dot(den0, tile[...], preferred_element_type=jnp.float32,
                    precision=lax.Precision.HIGHEST)
    dent1 = jnp.dot(den1, tile[...], preferred_element_type=jnp.float32,
                    precision=lax.Precision.HIGHEST)
    agg = jnp.concatenate([aref[0] / dent0, aref[1] / dent1], axis=1)
    res = agg + jnp.dot(nf[...], ws[...], preferred_element_type=jnp.float32) + bs[...]
    out[...] = jnp.dot(res, perm[...], preferred_element_type=jnp.float32,
                       precision=lax.Precision.HIGHEST)


def _finalize(agg, den, nf, ws, bs):
    tile = jnp.concatenate([jnp.eye(4, dtype=jnp.float32)] * (DH // 4), axis=1)
    perm = _perm_cols(jnp.eye(D_OUT, dtype=jnp.float32).T).T
    grid = (N_NODES // NODE_BLK,)
    row = pl.BlockSpec((NODE_BLK, D_OUT), lambda i: (i, 0))
    aspec = pl.BlockSpec((NC, NODE_BLK, DH), lambda i: (0, i, 0))
    dspec = pl.BlockSpec((NC, NS, NODE_BLK, 4), lambda i: (0, 0, i, 0))
    return pl.pallas_call(
        _final_body,
        grid=grid,
        in_specs=[aspec, dspec, row,
                  pl.BlockSpec((D_OUT, D_OUT), lambda i: (0, 0)),
                  pl.BlockSpec((1, D_OUT), lambda i: (0, 0)),
                  pl.BlockSpec((4, DH), lambda i: (0, 0)),
                  pl.BlockSpec((D_OUT, D_OUT), lambda i: (0, 0))],
        out_specs=row,
        out_shape=jax.ShapeDtypeStruct((N_NODES, D_OUT), jnp.float32),
    )(agg, den, nf, ws, bs, tile, perm)


def kernel(edge_tuples, edge_feats, edge_times_rel, node_feats, w_time, b_time,
           W_q, b_q, W_k, b_k, W_v, b_v, W_e, W_skip, b_skip):
    src = edge_tuples[0]
    dst = edge_tuples[1]

    wq = _split_cores(_perm_cols(W_q))
    wk = _split_cores(_perm_cols(W_k))
    wv = _split_cores(_perm_cols(W_v))
    we = _perm_cols(W_e)
    wsk = _perm_cols(W_skip)
    bq = _split_cores(_perm_cols(b_q.reshape(1, D_OUT)))
    bk = _split_cores(_perm_cols(b_k.reshape(1, D_OUT)))
    bv = _split_cores(_perm_cols(b_v.reshape(1, D_OUT)))
    bsk = _perm_cols(b_skip.reshape(1, D_OUT))

    q, k, v = _node_proj(node_feats, wq, bq, wk, bk, wv, bv)
    inv2pi = jnp.float32(1.0 / (2.0 * jnp.pi))
    e0, e1 = _edge_proj(edge_times_rel.reshape(E_TOT, 1), edge_feats,
                        w_time * inv2pi, b_time.reshape(1, T_DIM) * inv2pi,
                        we[:T_DIM], we[T_DIM:])
    agg, den = _sc_edge_pass(q, k, v, e0, e1, src, dst)
    den4 = den.reshape(NC, NS, N_NODES, 4)
    return _finalize(agg, den4, node_feats, wsk, bsk)


# speedup vs baseline: 28.0643x; 1.1702x over previous
import jax
import jax.numpy as jnp
from jax import lax
from jax.experimental import pallas as pl
from jax.experimental.pallas import tpu as pltpu
from jax.experimental.pallas import tpu_sc as plsc

N_NODES = 10000
E_TOT = 320000
HEADS = 8
D_HEAD = 16
D_OUT = 128
T_DIM = 32
D_EDGE = 16
DH = 64

NC = 2
NS = 16
NW = NC * NS
EPT = E_TOT // NS
EB = 80
NBATCH = EPT // EB
RPS = 624
REM = N_NODES - NS * RPS
N4 = N_NODES * 4

NODE_BLK = 400
EDGE_BLK = 2000


def _perm_cols(w):
    s = w.shape[:-1]
    w4 = w.reshape(*s, 2, 4, D_HEAD)
    w4 = jnp.swapaxes(w4, -1, -2)
    return w4.reshape(*s, D_OUT)


def _split_cores(w):
    k = w.shape[0]
    return w.reshape(k, NC, DH).swapaxes(0, 1)


def _node_proj_body(nf, wq, bq, wk, bk, wv, bv, q, k, v):
    x = nf[...]
    q[...] = jnp.dot(x, wq[0], preferred_element_type=jnp.float32) + bq[0]
    k[...] = jnp.dot(x, wk[0], preferred_element_type=jnp.float32) + bk[0]
    v[...] = jnp.dot(x, wv[0], preferred_element_type=jnp.float32) + bv[0]


def _node_proj(nf, wq, bq, wk, bk, wv, bv):
    grid = (NC, N_NODES // NODE_BLK)
    xrow = pl.BlockSpec((NODE_BLK, D_OUT), lambda c, i: (i, 0))
    wspec = pl.BlockSpec((1, D_OUT, DH), lambda c, i: (c, 0, 0))
    bspec = pl.BlockSpec((1, 1, DH), lambda c, i: (c, 0, 0))
    orow = pl.BlockSpec((NODE_BLK, DH),
                        lambda c, i: (c * (N_NODES // NODE_BLK) + i, 0))
    out = jax.ShapeDtypeStruct((NC * N_NODES, DH), jnp.float32)
    return pl.pallas_call(
        _node_proj_body,
        grid=grid,
        in_specs=[xrow, wspec, bspec, wspec, bspec, wspec, bspec],
        out_specs=[orow, orow, orow],
        out_shape=[out, out, out],
    )(nf, wq, bq, wk, bk, wv, bv)


_COSP = (0.9999999922898466, -19.73920555348366, 64.93917219630474,
         -85.45116501827795, 60.176223171395506, -26.000498057793394,
         6.575565933423059)


def _edge_proj_body(t, ef, w2, b2, wet, wef, e):
    u = w2[...] * t[0] + b2[...]
    u = u - jnp.round(u)
    z = u * u
    tf = jnp.float32(_COSP[6])
    for cc in _COSP[5::-1]:
        tf = tf * z + jnp.float32(cc)
    e[...] = (lax.dot_general(tf, wet[...], (((0,), (0,)), ((), ())),
                              preferred_element_type=jnp.float32)
              + jnp.dot(ef[...], wef[...], preferred_element_type=jnp.float32))


def _edge_proj(t2, ef, w2, b2, wet, wef):
    grid = (E_TOT // EDGE_BLK,)
    return pl.pallas_call(
        _edge_proj_body,
        grid=grid,
        in_specs=[
            pl.BlockSpec((1, 1, EDGE_BLK), lambda i: (i, 0, 0)),
            pl.BlockSpec((EDGE_BLK, D_EDGE), lambda i: (i, 0)),
            pl.BlockSpec((T_DIM, 1), lambda i: (0, 0)),
            pl.BlockSpec((T_DIM, 1), lambda i: (0, 0)),
            pl.BlockSpec((T_DIM, D_OUT), lambda i: (0, 0)),
            pl.BlockSpec((D_EDGE, D_OUT), lambda i: (0, 0)),
        ],
        out_specs=pl.BlockSpec((EDGE_BLK, D_OUT), lambda i: (i, 0)),
        out_shape=jax.ShapeDtypeStruct((E_TOT, D_OUT), jnp.float32),
    )(t2, ef, w2, b2, wet, wef)


def _sc_edge_body(q_hbm, k_hbm, v_hbm, e_hbm, src_hbm, dst_hbm,
                  agg_hbm, den_hbm,
                  src_v, dst_v, dstt_v, qb, kb, vb, eb,
                  src_v2, dst_v2, dstt_v2, qb2, kb2, vb2, eb2,
                  wb, agg_sh, den_sh, sem, sem2):
    c = lax.axis_index("c")
    s = lax.axis_index("s")
    lane16 = lax.iota(jnp.int32, 16)
    zero16 = jnp.zeros((16,), jnp.float32)

    zr2 = lane16 >> 3
    zc2 = lane16 & 7

    def zwb_body(i, carry):
        plsc.store_scatter(wb, [i * 2 + zr2, zc2], zero16)
        return carry

    lax.fori_loop(0, 320, zwb_body, 0)

    def zkb_body(i, carry):
        for jj in range(DH // 16):
            kb[i, pl.ds(16 * jj, 16)] = zero16
        return carry

    lax.fori_loop(0, EB, zkb_body, 0)

    r0 = s * RPS
    rtail = NS * RPS
    for i in range(7):
        pltpu.sync_copy(kb.at[pl.ds(0, EB)], agg_sh.at[pl.ds(r0 + i * EB, EB)])
    pltpu.sync_copy(kb.at[pl.ds(0, RPS - 7 * EB)],
                    agg_sh.at[pl.ds(r0 + 7 * EB, RPS - 7 * EB)])

    pltpu.sync_copy(wb.at[pl.ds(0, RPS)], den_sh.at[pl.ds(r0, RPS)])

    @pl.when(s == NS - 1)
    def _zero_tail():
        pltpu.sync_copy(kb.at[pl.ds(0, REM)], agg_sh.at[pl.ds(rtail, REM)])
        pltpu.sync_copy(wb.at[pl.ds(0, REM)], den_sh.at[pl.ds(rtail, REM)])

    plsc.subcore_barrier()

    base = s * EPT
    nbase = c * N_NODES
    rot8 = (lane16 + 8) & 15
    rot4 = (lane16 + 4) & 15
    low4 = lane16 < 4

    def make_group_body(qb, kb, vb, eb, dst_v, ecol):
        def group_body(g, carry):
            dvec = dst_v[pl.ds(g * 16, 16)]
            for l in range(16):
                b = g * 16 + l
                acc = jnp.zeros((16,), jnp.float32)
                for j in range(DH // 16):
                    sl = pl.ds(16 * j, 16)
                    acc = acc + qb[b, sl] * (kb[b, sl] + eb[b, pl.ds(ecol + 16 * j, 16)])
                a = acc + lax.gather(
                    acc, rot8.reshape(16, 1),
                    dimension_numbers=lax.GatherDimensionNumbers(
                        offset_dims=(), collapsed_slice_dims=(0,),
                        start_index_map=(0,)),
                    slice_sizes=(1,),
                    mode=lax.GatherScatterMode.PROMISE_IN_BOUNDS)
                a = a + lax.gather(
                    a, rot4.reshape(16, 1),
                    dimension_numbers=lax.GatherDimensionNumbers(
                        offset_dims=(), collapsed_slice_dims=(0,),
                        start_index_map=(0,)),
                    slice_sizes=(1,),
                    mode=lax.GatherScatterMode.PROMISE_IN_BOUNDS)
                w = jnp.exp(a * 0.25)
                for j in range(DH // 16):
                    sl = pl.ds(16 * j, 16)
                    vb[b, sl] = (vb[b, sl] + eb[b, pl.ds(ecol + 16 * j, 16)]) * w
                rows = jnp.where(low4, b, 632)
                plsc.store_scatter(wb, [rows, lane16 & 7], w)
            return carry
        return group_body

    bufsets = ((src_v, dst_v, dstt_v, qb, kb, vb, eb, sem),
               (src_v2, dst_v2, dstt_v2, qb2, kb2, vb2, eb2, sem2))

    def fire(i, bs):
        sv, dv, dtv, qx, kx, vx, ex, sm = bufsets[bs]
        off = base + i * EB
        pltpu.sync_copy(src_hbm.at[pl.ds(off, EB)], sv)
        pltpu.sync_copy(dst_hbm.at[pl.ds(off, EB)], dv)

        def idxoff_body(g, carry):
            sl = pl.ds(g * 16, 16)
            sv[sl] = sv[sl] + nbase
            dtv[sl] = dv[sl] + nbase
            return carry

        lax.fori_loop(0, EB // 16, idxoff_body, 0)
        pltpu.async_copy(q_hbm.at[dtv], qx, sm)
        pltpu.async_copy(k_hbm.at[sv], kx, sm)
        pltpu.async_copy(v_hbm.at[sv], vx, sm)
        pltpu.async_copy(e_hbm.at[pl.ds(off, EB)], ex, sm)

    def consume(i, bs):
        sv, dv, dtv, qx, kx, vx, ex, sm = bufsets[bs]
        pltpu.make_async_copy(q_hbm.at[dtv], qx, sm).wait()
        pltpu.make_async_copy(k_hbm.at[sv], kx, sm).wait()
        pltpu.make_async_copy(v_hbm.at[sv], vx, sm).wait()
        pltpu.make_async_copy(e_hbm.at[pl.ds(0, EB)], ex, sm).wait()

        @pl.when(c == 0)
        def _compute0():
            lax.fori_loop(0, EB // 16, make_group_body(qx, kx, vx, ex, dv, 0), 0)

        @pl.when(c == 1)
        def _compute1():
            lax.fori_loop(0, EB // 16, make_group_body(qx, kx, vx, ex, dv, DH), 0)

        pltpu.sync_copy(vx, agg_sh.at[dv], add=True)
        pltpu.sync_copy(wb.at[pl.ds(0, EB)], den_sh.at[dv], add=True)

    fire(0, 0)

    def pair_body(p, carry):
        fire(2 * p + 1, 1)
        consume(2 * p, 0)

        @pl.when(p + 1 < NBATCH // 2)
        def _fire_next():
            fire(2 * p + 2, 0)

        consume(2 * p + 1, 1)
        return carry

    lax.fori_loop(0, NBATCH // 2, pair_body, 0)

    plsc.subcore_barrier()
    pltpu.sync_copy(agg_sh.at[pl.ds(r0, RPS)], agg_hbm.at[c, pl.ds(r0, RPS)])

    pltpu.sync_copy(den_sh.at[pl.ds(r0, RPS)], den_hbm.at[c, pl.ds(r0, RPS)])

    @pl.when(s == NS - 1)
    def _drain_tail():
        pltpu.sync_copy(agg_sh.at[pl.ds(rtail, REM)], agg_hbm.at[c, pl.ds(rtail, REM)])
        pltpu.sync_copy(den_sh.at[pl.ds(rtail, REM)], den_hbm.at[c, pl.ds(rtail, REM)])


def _sc_edge_pass(q, k, v, e, src, dst):
    mesh = plsc.VectorSubcoreMesh(core_axis_name="c", subcore_axis_name="s")
    call = pl.kernel(
        _sc_edge_body,
        out_type=[
            jax.ShapeDtypeStruct((NC, N_NODES, DH), jnp.float32),
            jax.ShapeDtypeStruct((NC, N_NODES, 8), jnp.float32),
        ],
        mesh=mesh,
        compiler_params=pltpu.CompilerParams(needs_layout_passes=False,
                                            use_tc_tiling_on_sc=False),
        scratch_types=[
            pltpu.VMEM((EB,), jnp.int32),
            pltpu.VMEM((EB,), jnp.int32),
            pltpu.VMEM((EB,), jnp.int32),
            pltpu.VMEM((EB, DH), jnp.float32),
            pltpu.VMEM((EB, DH), jnp.float32),
            pltpu.VMEM((EB, DH), jnp.float32),
            pltpu.VMEM((EB, D_OUT), jnp.float32),
            pltpu.VMEM((EB,), jnp.int32),
            pltpu.VMEM((EB,), jnp.int32),
            pltpu.VMEM((EB,), jnp.int32),
            pltpu.VMEM((EB, DH), jnp.float32),
            pltpu.VMEM((EB, DH), jnp.float32),
            pltpu.VMEM((EB, DH), jnp.float32),
            pltpu.VMEM((EB, D_OUT), jnp.float32),
            pltpu.VMEM((640, 8), jnp.float32),
            pltpu.VMEM_SHARED((N_NODES, DH), jnp.float32),
            pltpu.VMEM_SHARED((N_NODES, 8), jnp.float32),
            pltpu.SemaphoreType.DMA,
            pltpu.SemaphoreType.DMA,
        ],
    )
    return call(q, k, v, e, src, dst)


def _final_body(aref, dref, nf, ws, bs, tile, perm, out):
    den0 = dref[0][:, :4] + 1e-16
    den1 = dref[1][:, :4] + 1e-16
    dent0 = jnp.dot(den0, tile[...], preferred_element_type=jnp.float32,
                    precision=lax.Precision.HIGHEST)
    dent1 = jnp.dot(den1, tile[...], preferred_element_type=jnp.float32,
                    precision=lax.Precision.HIGHEST)
    agg = jnp.concatenate([aref[0] / dent0, aref[1] / dent1], axis=1)
    res = agg + jnp.dot(nf[...], ws[...], preferred_element_type=jnp.float32) + bs[...]
    out[...] = jnp.dot(res, perm[...], preferred_element_type=jnp.float32,
                       precision=lax.Precision.HIGHEST)


def _finalize(agg, den, nf, ws, bs):
    tile = jnp.concatenate([jnp.eye(4, dtype=jnp.float32)] * (DH // 4), axis=1)
    perm = _perm_cols(jnp.eye(D_OUT, dtype=jnp.float32).T).T
    grid = (N_NODES // NODE_BLK,)
    row = pl.BlockSpec((NODE_BLK, D_OUT), lambda i: (i, 0))
    aspec = pl.BlockSpec((NC, NODE_BLK, DH), lambda i: (0, i, 0))
    dspec = pl.BlockSpec((NC, NODE_BLK, 8), lambda i: (0, i, 0))
    return pl.pallas_call(
        _final_body,
        grid=grid,
        in_specs=[aspec, dspec, row,
                  pl.BlockSpec((D_OUT, D_OUT), lambda i: (0, 0)),
                  pl.BlockSpec((1, D_OUT), lambda i: (0, 0)),
                  pl.BlockSpec((4, DH), lambda i: (0, 0)),
                  pl.BlockSpec((D_OUT, D_OUT), lambda i: (0, 0))],
        out_specs=row,
        out_shape=jax.ShapeDtypeStruct((N_NODES, D_OUT), jnp.float32),
    )(agg, den, nf, ws, bs, tile, perm)


def kernel(edge_tuples, edge_feats, edge_times_rel, node_feats, w_time, b_time,
           W_q, b_q, W_k, b_k, W_v, b_v, W_e, W_skip, b_skip):
    src = edge_tuples[0]
    dst = edge_tuples[1]

    wq = _split_cores(_perm_cols(W_q))
    wk = _split_cores(_perm_cols(W_k))
    wv = _split_cores(_perm_cols(W_v))
    we = _perm_cols(W_e)
    wsk = _perm_cols(W_skip)
    bq = _split_cores(_perm_cols(b_q.reshape(1, D_OUT)))
    bk = _split_cores(_perm_cols(b_k.reshape(1, D_OUT)))
    bv = _split_cores(_perm_cols(b_v.reshape(1, D_OUT)))
    bsk = _perm_cols(b_skip.reshape(1, D_OUT))

    q, k, v = _node_proj(node_feats, wq, bq, wk, bk, wv, bv)
    inv2pi = jnp.float32(1.0 / (2.0 * jnp.pi))
    e = _edge_proj(edge_times_rel.reshape(E_TOT // EDGE_BLK, 1, EDGE_BLK),
                   edge_feats,
                   (w_time * inv2pi).reshape(T_DIM, 1),
                   (b_time * inv2pi).reshape(T_DIM, 1),
                   we[:T_DIM], we[T_DIM:])
    agg, den = _sc_edge_pass(q, k, v, e, src, dst)
    return _finalize(agg, den, node_feats, wsk, bsk)


# speedup vs baseline: 35.6184x; 1.2692x over previous
import jax
import jax.numpy as jnp
from jax import lax
from jax.experimental import pallas as pl
from jax.experimental.pallas import tpu as pltpu
from jax.experimental.pallas import tpu_sc as plsc

N_NODES = 10000
E_TOT = 320000
HEADS = 8
D_HEAD = 16
D_OUT = 128
T_DIM = 32
D_EDGE = 16
DH = 64

NC = 2
NS = 16
NW = NC * NS
EPT = E_TOT // NS
EB = 80
NBATCH = EPT // EB
RPS = 624
REM = N_NODES - NS * RPS
N4 = N_NODES * 4

NODE_BLK = 400
EDGE_BLK = 2000


def _perm_cols(w):
    s = w.shape[:-1]
    w4 = w.reshape(*s, 2, 4, D_HEAD)
    w4 = jnp.swapaxes(w4, -1, -2)
    return w4.reshape(*s, D_OUT)


def _split_cores(w):
    k = w.shape[0]
    return w.reshape(k, NC, DH).swapaxes(0, 1)


def _node_proj_body(nf, wq, bq, wk, bk, wv, bv, q, k, v):
    x = nf[...]
    q[...] = jnp.dot(x, wq[0], preferred_element_type=jnp.float32) + bq[0]
    k[...] = jnp.dot(x, wk[0], preferred_element_type=jnp.float32) + bk[0]
    v[...] = jnp.dot(x, wv[0], preferred_element_type=jnp.float32) + bv[0]


def _node_proj(nf, wq, bq, wk, bk, wv, bv):
    grid = (NC, N_NODES // NODE_BLK)
    xrow = pl.BlockSpec((NODE_BLK, D_OUT), lambda c, i: (i, 0))
    wspec = pl.BlockSpec((1, D_OUT, DH), lambda c, i: (c, 0, 0))
    bspec = pl.BlockSpec((1, 1, DH), lambda c, i: (c, 0, 0))
    orow = pl.BlockSpec((NODE_BLK, DH),
                        lambda c, i: (c * (N_NODES // NODE_BLK) + i, 0))
    out = jax.ShapeDtypeStruct((NC * N_NODES, DH), jnp.float32)
    return pl.pallas_call(
        _node_proj_body,
        grid=grid,
        in_specs=[xrow, wspec, bspec, wspec, bspec, wspec, bspec],
        out_specs=[orow, orow, orow],
        out_shape=[out, out, out],
    )(nf, wq, bq, wk, bk, wv, bv)


_COSP = (0.9999999922898466, -19.73920555348366, 64.93917219630474,
         -85.45116501827795, 60.176223171395506, -26.000498057793394,
         6.575565933423059)


def _edge_proj_body(t, ef, w2, b2, wet, wef, e):
    u = w2[...] * t[0] + b2[...]
    u = u - jnp.round(u)
    z = u * u
    tf = jnp.float32(_COSP[6])
    for cc in _COSP[5::-1]:
        tf = tf * z + jnp.float32(cc)
    e[...] = (lax.dot_general(tf, wet[...], (((0,), (0,)), ((), ())),
                              preferred_element_type=jnp.float32)
              + jnp.dot(ef[...], wef[...], preferred_element_type=jnp.float32))


def _edge_proj(t2, ef, w2, b2, wet, wef):
    grid = (E_TOT // EDGE_BLK,)
    return pl.pallas_call(
        _edge_proj_body,
        grid=grid,
        in_specs=[
            pl.BlockSpec((1, 1, EDGE_BLK), lambda i: (i, 0, 0)),
            pl.BlockSpec((EDGE_BLK, D_EDGE), lambda i: (i, 0)),
            pl.BlockSpec((T_DIM, 1), lambda i: (0, 0)),
            pl.BlockSpec((T_DIM, 1), lambda i: (0, 0)),
            pl.BlockSpec((T_DIM, D_OUT), lambda i: (0, 0)),
            pl.BlockSpec((D_EDGE, D_OUT), lambda i: (0, 0)),
        ],
        out_specs=pl.BlockSpec((EDGE_BLK, D_OUT), lambda i: (i, 0)),
        out_shape=jax.ShapeDtypeStruct((E_TOT, D_OUT), jnp.float32),
    )(t2, ef, w2, b2, wet, wef)


def _sc_edge_body(q_hbm, k_hbm, v_hbm, e_hbm, src_hbm, dst_hbm,
                  agg_hbm, den_hbm,
                  src_v, dst_v, dstt_v, qb, kb, vb, eb,
                  src_v2, dst_v2, dstt_v2, qb2, kb2, vb2, eb2,
                  wb, agg_sh, den_sh, sem, sem2):
    c = lax.axis_index("c")
    s = lax.axis_index("s")
    lane16 = lax.iota(jnp.int32, 16)
    zero16 = jnp.zeros((16,), jnp.float32)

    zr2 = lane16 >> 3
    zc2 = lane16 & 7

    def zwb_body(i, carry):
        plsc.store_scatter(wb, [i * 2 + zr2, zc2], zero16)
        return carry

    lax.fori_loop(0, 320, zwb_body, 0)

    def zkb_body(i, carry):
        for jj in range(DH // 16):
            kb[i, pl.ds(16 * jj, 16)] = zero16
        return carry

    lax.fori_loop(0, EB, zkb_body, 0)

    r0 = s * RPS
    rtail = NS * RPS
    for i in range(7):
        pltpu.sync_copy(kb.at[pl.ds(0, EB)], agg_sh.at[pl.ds(r0 + i * EB, EB)])
    pltpu.sync_copy(kb.at[pl.ds(0, RPS - 7 * EB)],
                    agg_sh.at[pl.ds(r0 + 7 * EB, RPS - 7 * EB)])

    pltpu.sync_copy(wb.at[pl.ds(0, RPS)], den_sh.at[pl.ds(r0, RPS)])

    @pl.when(s == NS - 1)
    def _zero_tail():
        pltpu.sync_copy(kb.at[pl.ds(0, REM)], agg_sh.at[pl.ds(rtail, REM)])
        pltpu.sync_copy(wb.at[pl.ds(0, REM)], den_sh.at[pl.ds(rtail, REM)])

    plsc.subcore_barrier()

    base = s * EPT
    nbase = c * N_NODES
    ecolbase = c * DH
    rot8 = (lane16 + 8) & 15
    rot4 = (lane16 + 4) & 15
    low4 = lane16 < 4

    def make_group_body(qb, kb, vb, eb, dst_v, ecol):
        def group_body(g, carry):
            dvec = dst_v[pl.ds(g * 16, 16)]
            for l in range(16):
                b = g * 16 + l
                acc = jnp.zeros((16,), jnp.float32)
                for j in range(DH // 16):
                    sl = pl.ds(16 * j, 16)
                    acc = acc + qb[b, sl] * (kb[b, sl] + eb[b, pl.ds(ecol + 16 * j, 16)])
                a = acc + lax.gather(
                    acc, rot8.reshape(16, 1),
                    dimension_numbers=lax.GatherDimensionNumbers(
                        offset_dims=(), collapsed_slice_dims=(0,),
                        start_index_map=(0,)),
                    slice_sizes=(1,),
                    mode=lax.GatherScatterMode.PROMISE_IN_BOUNDS)
                a = a + lax.gather(
                    a, rot4.reshape(16, 1),
                    dimension_numbers=lax.GatherDimensionNumbers(
                        offset_dims=(), collapsed_slice_dims=(0,),
                        start_index_map=(0,)),
                    slice_sizes=(1,),
                    mode=lax.GatherScatterMode.PROMISE_IN_BOUNDS)
                w = jnp.exp(a * 0.25)
                for j in range(DH // 16):
                    sl = pl.ds(16 * j, 16)
                    vb[b, sl] = (vb[b, sl] + eb[b, pl.ds(ecol + 16 * j, 16)]) * w
                rows = jnp.where(low4, b, 632)
                plsc.store_scatter(wb, [rows, lane16 & 7], w)
            return carry
        return group_body

    bufsets = ((src_v, dst_v, dstt_v, qb, kb, vb, eb, sem),
               (src_v2, dst_v2, dstt_v2, qb2, kb2, vb2, eb2, sem2))

    def fire(i, bs):
        sv, dv, dtv, qx, kx, vx, ex, sm = bufsets[bs]
        off = base + i * EB
        pltpu.sync_copy(src_hbm.at[pl.ds(off, EB)], sv)
        pltpu.sync_copy(dst_hbm.at[pl.ds(off, EB)], dv)

        def idxoff_body(g, carry):
            sl = pl.ds(g * 16, 16)
            sv[sl] = sv[sl] + nbase
            dtv[sl] = dv[sl] + nbase
            return carry

        lax.fori_loop(0, EB // 16, idxoff_body, 0)
        pltpu.async_copy(q_hbm.at[dtv], qx, sm)
        pltpu.async_copy(k_hbm.at[sv], kx, sm)
        pltpu.async_copy(v_hbm.at[sv], vx, sm)
        pltpu.async_copy(e_hbm.at[pl.ds(off, EB), pl.ds(ecolbase, DH)], ex, sm)

    def consume(i, bs):
        sv, dv, dtv, qx, kx, vx, ex, sm = bufsets[bs]
        pltpu.make_async_copy(q_hbm.at[dtv], qx, sm).wait()
        pltpu.make_async_copy(k_hbm.at[sv], kx, sm).wait()
        pltpu.make_async_copy(v_hbm.at[sv], vx, sm).wait()
        pltpu.make_async_copy(e_hbm.at[pl.ds(0, EB), pl.ds(0, DH)], ex, sm).wait()

        lax.fori_loop(0, EB // 16, make_group_body(qx, kx, vx, ex, dv, 0), 0)

        pltpu.sync_copy(vx, agg_sh.at[dv], add=True)
        pltpu.sync_copy(wb.at[pl.ds(0, EB)], den_sh.at[dv], add=True)

    fire(0, 0)

    def pair_body(p, carry):
        fire(2 * p + 1, 1)
        consume(2 * p, 0)

        @pl.when(p + 1 < NBATCH // 2)
        def _fire_next():
            fire(2 * p + 2, 0)

        consume(2 * p + 1, 1)
        return carry

    lax.fori_loop(0, NBATCH // 2, pair_body, 0)

    plsc.subcore_barrier()
    pltpu.sync_copy(agg_sh.at[pl.ds(r0, RPS)], agg_hbm.at[c, pl.ds(r0, RPS)])

    pltpu.sync_copy(den_sh.at[pl.ds(r0, RPS)], den_hbm.at[c, pl.ds(r0, RPS)])

    @pl.when(s == NS - 1)
    def _drain_tail():
        pltpu.sync_copy(agg_sh.at[pl.ds(rtail, REM)], agg_hbm.at[c, pl.ds(rtail, REM)])
        pltpu.sync_copy(den_sh.at[pl.ds(rtail, REM)], den_hbm.at[c, pl.ds(rtail, REM)])


def _sc_edge_pass(q, k, v, e, src, dst):
    mesh = plsc.VectorSubcoreMesh(core_axis_name="c", subcore_axis_name="s")
    call = pl.kernel(
        _sc_edge_body,
        out_type=[
            jax.ShapeDtypeStruct((NC, N_NODES, DH), jnp.float32),
            jax.ShapeDtypeStruct((NC, N_NODES, 8), jnp.float32),
        ],
        mesh=mesh,
        compiler_params=pltpu.CompilerParams(needs_layout_passes=False,
                                            use_tc_tiling_on_sc=False),
        scratch_types=[
            pltpu.VMEM((EB,), jnp.int32),
            pltpu.VMEM((EB,), jnp.int32),
            pltpu.VMEM((EB,), jnp.int32),
            pltpu.VMEM((EB, DH), jnp.float32),
            pltpu.VMEM((EB, DH), jnp.float32),
            pltpu.VMEM((EB, DH), jnp.float32),
            pltpu.VMEM((EB, DH), jnp.float32),
            pltpu.VMEM((EB,), jnp.int32),
            pltpu.VMEM((EB,), jnp.int32),
            pltpu.VMEM((EB,), jnp.int32),
            pltpu.VMEM((EB, DH), jnp.float32),
            pltpu.VMEM((EB, DH), jnp.float32),
            pltpu.VMEM((EB, DH), jnp.float32),
            pltpu.VMEM((EB, DH), jnp.float32),
            pltpu.VMEM((640, 8), jnp.float32),
            pltpu.VMEM_SHARED((N_NODES, DH), jnp.float32),
            pltpu.VMEM_SHARED((N_NODES, 8), jnp.float32),
            pltpu.SemaphoreType.DMA,
            pltpu.SemaphoreType.DMA,
        ],
    )
    return call(q, k, v, e, src, dst)


def _final_body(aref, dref, nf, ws, bs, tile, perm, out):
    den0 = dref[0][:, :4] + 1e-16
    den1 = dref[1][:, :4] + 1e-16
    dent0 = jnp.dot(den0, tile[...], preferred_element_type=jnp.float32,
                    precision=lax.Precision.HIGHEST)
    dent1 = jnp.dot(den1, tile[...], preferred_element_type=jnp.float32,
                    precision=lax.Precision.HIGHEST)
    agg = jnp.concatenate([aref[0] / dent0, aref[1] / dent1], axis=1)
    res = agg + jnp.dot(nf[...], ws[...], preferred_element_type=jnp.float32) + bs[...]
    out[...] = jnp.dot(res, perm[...], preferred_element_type=jnp.float32,
                       precision=lax.Precision.HIGHEST)


def _finalize(agg, den, nf, ws, bs):
    tile = jnp.concatenate([jnp.eye(4, dtype=jnp.float32)] * (DH // 4), axis=1)
    perm = _perm_cols(jnp.eye(D_OUT, dtype=jnp.float32).T).T
    grid = (N_NODES // NODE_BLK,)
    row = pl.BlockSpec((NODE_BLK, D_OUT), lambda i: (i, 0))
    aspec = pl.BlockSpec((NC, NODE_BLK, DH), lambda i: (0, i, 0))
    dspec = pl.BlockSpec((NC, NODE_BLK, 8), lambda i: (0, i, 0))
    return pl.pallas_call(
        _final_body,
        grid=grid,
        in_specs=[aspec, dspec, row,
                  pl.BlockSpec((D_OUT, D_OUT), lambda i: (0, 0)),
                  pl.BlockSpec((1, D_OUT), lambda i: (0, 0)),
                  pl.BlockSpec((4, DH), lambda i: (0, 0)),
                  pl.BlockSpec((D_OUT, D_OUT), lambda i: (0, 0))],
        out_specs=row,
        out_shape=jax.ShapeDtypeStruct((N_NODES, D_OUT), jnp.float32),
    )(agg, den, nf, ws, bs, tile, perm)


def kernel(edge_tuples, edge_feats, edge_times_rel, node_feats, w_time, b_time,
           W_q, b_q, W_k, b_k, W_v, b_v, W_e, W_skip, b_skip):
    src = edge_tuples[0]
    dst = edge_tuples[1]

    wq = _split_cores(_perm_cols(W_q))
    wk = _split_cores(_perm_cols(W_k))
    wv = _split_cores(_perm_cols(W_v))
    we = _perm_cols(W_e)
    wsk = _perm_cols(W_skip)
    bq = _split_cores(_perm_cols(b_q.reshape(1, D_OUT)))
    bk = _split_cores(_perm_cols(b_k.reshape(1, D_OUT)))
    bv = _split_cores(_perm_cols(b_v.reshape(1, D_OUT)))
    bsk = _perm_cols(b_skip.reshape(1, D_OUT))

    q, k, v = _node_proj(node_feats, wq, bq, wk, bk, wv, bv)
    inv2pi = jnp.float32(1.0 / (2.0 * jnp.pi))
    e = _edge_proj(edge_times_rel.reshape(E_TOT // EDGE_BLK, 1, EDGE_BLK),
                   edge_feats,
                   (w_time * inv2pi).reshape(T_DIM, 1),
                   (b_time * inv2pi).reshape(T_DIM, 1),
                   we[:T_DIM], we[T_DIM:])
    agg, den = _sc_edge_pass(q, k, v, e, src, dst)
    return _finalize(agg, den, node_feats, wsk, bsk)
